# baseline (speedup 1.0000x reference)
import functools

import jax
import jax.numpy as jnp
from jax import lax
from jax.experimental import pallas as pl
from jax.experimental.pallas import tpu as pltpu
from jax.experimental.pallas import tpu_sc as plsc

_N = 10000
_NP = 10240
_E = 320000
_EPAD = 327680
_NC = 2
_NS = 16
_CHUNK = 128
_DH = 32
_RPT = _NP // _NS
_NCH = _EPAD // _NS // _CHUNK
_DEG_NCH = _EPAD // (_NC * _NS) // _CHUNK

_sc_mesh = plsc.VectorSubcoreMesh(core_axis_name="c", subcore_axis_name="s")
_sc_params = pltpu.CompilerParams(use_tc_tiling_on_sc=False,
                                  needs_layout_passes=False)


@functools.partial(
    pl.kernel,
    mesh=_sc_mesh,
    compiler_params=_sc_params,
    out_type=jax.ShapeDtypeStruct((_NC, _NP, 16), jnp.float32),
    scratch_types=[
        pltpu.VMEM((_DEG_NCH, _CHUNK), jnp.int32),
        pltpu.VMEM((_CHUNK, 16), jnp.float32),
        pltpu.VMEM_SHARED((_NP, 16), jnp.float32),
    ],
)
def _degrees(ones_hbm, col_hbm, out_hbm, idxc, rows, acc):
    c = lax.axis_index("c")
    s = lax.axis_index("s")
    wid = c * _NS + s
    pltpu.sync_copy(ones_hbm.at[pl.ds(s * _RPT, _RPT)],
                    acc.at[pl.ds(s * _RPT, _RPT)])
    pltpu.sync_copy(col_hbm.at[pl.ds(wid * _DEG_NCH, _DEG_NCH)], idxc)
    pltpu.sync_copy(ones_hbm.at[pl.ds(0, _CHUNK)], rows)
    plsc.subcore_barrier()

    def body(i, carry):
        pltpu.sync_copy(rows, acc.at[idxc.at[i]], add=True)
        return carry

    lax.fori_loop(0, _DEG_NCH, body, 0)
    plsc.subcore_barrier()
    pltpu.sync_copy(acc.at[pl.ds(s * _RPT, _RPT)],
                    out_hbm.at[c, pl.ds(s * _RPT, _RPT)])


def _hop_pipeline(src, idxr, idxc, rows_a, rows_b, dst, sga, sgb, ssa, ssb):
    pltpu.async_copy(src.at[idxr.at[0]], rows_a, sga)
    pltpu.async_copy(src.at[idxr.at[1]], rows_b, sgb)

    def body(g, carry):
        pltpu.make_async_copy(src.at[idxr.at[0]], rows_a, sga).wait()
        pltpu.async_copy(rows_a, dst.at[idxc.at[2 * g]], ssa, add=True)
        pltpu.make_async_copy(src.at[idxr.at[1]], rows_b, sgb).wait()
        pltpu.async_copy(rows_b, dst.at[idxc.at[2 * g + 1]], ssb, add=True)
        pltpu.make_async_copy(rows_a, dst.at[idxc.at[0]], ssa).wait()
        pltpu.async_copy(src.at[idxr.at[2 * g + 2]], rows_a, sga)
        pltpu.make_async_copy(rows_b, dst.at[idxc.at[0]], ssb).wait()
        pltpu.async_copy(src.at[idxr.at[2 * g + 3]], rows_b, sgb)
        return carry

    lax.fori_loop(0, _NCH // 2 - 1, body, 0)
    pltpu.make_async_copy(src.at[idxr.at[0]], rows_a, sga).wait()
    pltpu.sync_copy(rows_a, dst.at[idxc.at[_NCH - 2]], add=True)
    pltpu.make_async_copy(src.at[idxr.at[1]], rows_b, sgb).wait()
    pltpu.sync_copy(rows_b, dst.at[idxc.at[_NCH - 1]], add=True)


@functools.partial(
    pl.kernel,
    mesh=_sc_mesh,
    compiler_params=_sc_params,
    out_type=jax.ShapeDtypeStruct((_NC, _NP, _DH), jnp.bfloat16),
    scratch_types=[
        pltpu.VMEM((_NCH, _CHUNK), jnp.int32),
        pltpu.VMEM((_NCH, _CHUNK), jnp.int32),
        pltpu.VMEM((_CHUNK, _DH), jnp.bfloat16),
        pltpu.VMEM((_CHUNK, _DH), jnp.bfloat16),
        pltpu.VMEM((_RPT, _DH), jnp.bfloat16),
        pltpu.VMEM((_RPT,), jnp.float32),
        pltpu.VMEM_SHARED((_NP, _DH), jnp.bfloat16),
        pltpu.VMEM_SHARED((_NP, _DH), jnp.bfloat16),
        pltpu.SemaphoreType.DMA,
        pltpu.SemaphoreType.DMA,
        pltpu.SemaphoreType.DMA,
        pltpu.SemaphoreType.DMA,
    ],
)
def _layer(u_hbm, dinv_hbm, row_hbm, col_hbm, out_hbm, idxr, idxc,
           rows_a, rows_b, stage, dv, acc1, acc2, sga, sgb, ssa, ssb):
    c = lax.axis_index("c")
    s = lax.axis_index("s")
    rb = s * _RPT
    pltpu.sync_copy(u_hbm.at[c, pl.ds(rb, _RPT)], acc1.at[pl.ds(rb, _RPT)])
    pltpu.sync_copy(row_hbm.at[pl.ds(s * _NCH, _NCH)], idxr)
    pltpu.sync_copy(col_hbm.at[pl.ds(s * _NCH, _NCH)], idxc)
    plsc.subcore_barrier()

    _hop_pipeline(u_hbm.at[c], idxr, idxc, rows_a, rows_b, acc1,
                  sga, sgb, ssa, ssb)
    plsc.subcore_barrier()

    pltpu.sync_copy(dinv_hbm.at[pl.ds(rb, _RPT)], dv)
    pltpu.sync_copy(acc1.at[pl.ds(rb, _RPT)], stage)

    def scale(m, carry):
        dvec = dv[pl.ds(m * 16, 16)]
        for j in range(16):
            n = m * 16 + j
            d2 = dvec[j] * dvec[j]
            va, vb = plsc.unpack(stage[n, :],
                                 format=plsc.PackFormat.INTERLEAVED)
            stage[n, :] = plsc.pack(va * d2, vb * d2,
                                    format=plsc.PackFormat.INTERLEAVED)
        return carry

    lax.fori_loop(0, _RPT // 16, scale, 0)
    pltpu.sync_copy(stage, acc1.at[pl.ds(rb, _RPT)])
    pltpu.sync_copy(stage, acc2.at[pl.ds(rb, _RPT)])
    plsc.subcore_barrier()

    _hop_pipeline(acc1, idxr, idxc, rows_a, rows_b, acc2,
                  sga, sgb, ssa, ssb)
    plsc.subcore_barrier()
    pltpu.sync_copy(acc2.at[pl.ds(rb, _RPT)], out_hbm.at[c, pl.ds(rb, _RPT)])


_ROWS_B = 1024
_GRID = _NP // _ROWS_B


def _entry_body(d0_ref, d1_ref, x_ref, w1_ref, dinv_ref, dinvf_ref, u_ref):
    deg = d0_ref[0, :, :1] + d1_ref[0, :, :1] - 1.0
    dinv = lax.rsqrt(deg)
    dinv_ref[:] = dinv
    dinvf_ref[:] = dinv.reshape(_ROWS_B // 128, 128)
    for c in range(_NC):
        u_ref[c] = (jnp.dot(x_ref[:], w1_ref[c],
                            preferred_element_type=jnp.float32)
                    * dinv).astype(jnp.bfloat16)


def _entry(degp, xp, W1):
    return pl.pallas_call(
        _entry_body,
        grid=(_GRID,),
        in_specs=[
            pl.BlockSpec((1, _ROWS_B, 16), lambda i: (0, i, 0)),
            pl.BlockSpec((1, _ROWS_B, 16), lambda i: (1, i, 0)),
            pl.BlockSpec((_ROWS_B, 128), lambda i: (i, 0)),
            pl.BlockSpec((_NC, 128, _DH), lambda i: (0, 0, 0)),
        ],
        out_specs=[
            pl.BlockSpec((_ROWS_B, 1), lambda i: (i, 0)),
            pl.BlockSpec((_ROWS_B // 128, 128), lambda i: (i, 0)),
            pl.BlockSpec((_NC, _ROWS_B, _DH), lambda i: (0, i, 0)),
        ],
        out_shape=[
            jax.ShapeDtypeStruct((_NP, 1), jnp.float32),
            jax.ShapeDtypeStruct((_NP // 128, 128), jnp.float32),
            jax.ShapeDtypeStruct((_NC, _NP, _DH), jnp.bfloat16),
        ],
    )(degp, degp, xp, W1)


def _mid_body(z_ref, dinv_ref, b1_ref, w2_ref, u_ref):
    dinv = dinv_ref[:]
    z = jnp.concatenate([z_ref[0], z_ref[1]],
                        axis=1).astype(jnp.float32)
    h = jnp.maximum(z * dinv + b1_ref[:], 0.0)
    for c in range(_NC):
        u_ref[c] = (jnp.dot(h, w2_ref[c],
                            preferred_element_type=jnp.float32)
                    * dinv).astype(jnp.bfloat16)


def _mid(z3, dinv, b1, W2p):
    return pl.pallas_call(
        _mid_body,
        grid=(_GRID,),
        in_specs=[
            pl.BlockSpec((_NC, _ROWS_B, _DH), lambda i: (0, i, 0)),
            pl.BlockSpec((_ROWS_B, 1), lambda i: (i, 0)),
            pl.BlockSpec((1, 2 * _DH), lambda i: (0, 0)),
            pl.BlockSpec((_NC, 2 * _DH, _DH), lambda i: (0, 0, 0)),
        ],
        out_specs=pl.BlockSpec((_NC, _ROWS_B, _DH), lambda i: (0, i, 0)),
        out_shape=jax.ShapeDtypeStruct((_NC, _NP, _DH), jnp.bfloat16),
    )(z3, dinv, b1, W2p)


def _final_body(z_ref, dinv_ref, b2_ref, out_ref):
    z = jnp.concatenate([z_ref[0], z_ref[1]],
                        axis=1).astype(jnp.float32)
    t = z * dinv_ref[:] + b2_ref[:]
    t = t[:, :40]
    m = jnp.max(t, axis=1, keepdims=True)
    e = t - m
    out_ref[:] = e - jnp.log(jnp.sum(jnp.exp(e), axis=1, keepdims=True))


def _final(z3, dinv, b2p):
    return pl.pallas_call(
        _final_body,
        grid=(_GRID,),
        in_specs=[
            pl.BlockSpec((_NC, _ROWS_B, _DH), lambda i: (0, i, 0)),
            pl.BlockSpec((_ROWS_B, 1), lambda i: (i, 0)),
            pl.BlockSpec((1, 2 * _DH), lambda i: (0, 0)),
        ],
        out_specs=pl.BlockSpec((_ROWS_B, 40), lambda i: (i, 0)),
        out_shape=jax.ShapeDtypeStruct((_NP, 40), jnp.float32),
    )(z3, dinv, b2p)


def kernel(x, edge_index, W1, b1, W2, b2):
    epad = _N + jnp.arange(_EPAD - _E, dtype=jnp.int32) % (_NP - _N)
    row = jnp.concatenate([edge_index[0], epad]).reshape(-1, _CHUNK)
    col = jnp.concatenate([edge_index[1], epad]).reshape(-1, _CHUNK)
    xp = jnp.pad(x, ((0, _NP - _N), (0, 0)))
    ones16 = jnp.ones((_NP, 16), jnp.float32)
    W2f = jnp.pad(W2, ((0, 0), (0, 2 * _DH - 40)))
    W1c = jnp.stack([W1[:, :_DH], W1[:, _DH:]])
    W2p = jnp.stack([W2f[:, :_DH], W2f[:, _DH:]])
    b1r = b1.reshape(1, 2 * _DH)
    b2r = jnp.pad(b2, (0, 2 * _DH - 40)).reshape(1, 2 * _DH)

    degp = _degrees(ones16, col)
    dinv, dinvf, u1 = _entry(degp, xp, W1c)
    dinv1 = dinvf.reshape(_NP)
    z1 = _layer(u1, dinv1, row, col)
    u2 = _mid(z1, dinv, b1r, W2p)
    z2 = _layer(u2, dinv1, row, col)
    out = _final(z2, dinv, b2r)
    return out[:_N]


# speedup vs baseline: 1.0607x; 1.0607x over previous
import functools

import jax
import jax.numpy as jnp
from jax import lax
from jax.experimental import pallas as pl
from jax.experimental.pallas import tpu as pltpu
from jax.experimental.pallas import tpu_sc as plsc

_N = 10000
_NP = 10240
_E = 320000
_EPAD = 327680
_NC = 2
_NS = 16
_CHUNK = 128
_DH = 32
_RPT = _NP // _NS
_NCH = _EPAD // _NS // _CHUNK
_DEG_NCH = _EPAD // (_NC * _NS) // _CHUNK

_sc_mesh = plsc.VectorSubcoreMesh(core_axis_name="c", subcore_axis_name="s")
_sc_params = pltpu.CompilerParams(use_tc_tiling_on_sc=False,
                                  needs_layout_passes=False)


@functools.partial(
    pl.kernel,
    mesh=_sc_mesh,
    compiler_params=_sc_params,
    out_type=jax.ShapeDtypeStruct((_NC, _NP, 16), jnp.float32),
    scratch_types=[
        pltpu.VMEM((_DEG_NCH, _CHUNK), jnp.int32),
        pltpu.VMEM((_CHUNK, 16), jnp.float32),
        pltpu.VMEM_SHARED((_NP, 16), jnp.float32),
    ],
)
def _degrees(ones_hbm, col_hbm, out_hbm, idxc, rows, acc):
    c = lax.axis_index("c")
    s = lax.axis_index("s")
    wid = c * _NS + s
    pltpu.sync_copy(ones_hbm.at[pl.ds(s * _RPT, _RPT)],
                    acc.at[pl.ds(s * _RPT, _RPT)])
    pltpu.sync_copy(col_hbm.at[pl.ds(wid * _DEG_NCH, _DEG_NCH)], idxc)
    pltpu.sync_copy(ones_hbm.at[pl.ds(0, _CHUNK)], rows)
    plsc.subcore_barrier()

    def body(i, carry):
        pltpu.sync_copy(rows, acc.at[idxc.at[i]], add=True)
        return carry

    lax.fori_loop(0, _DEG_NCH, body, 0)
    plsc.subcore_barrier()
    pltpu.sync_copy(acc.at[pl.ds(s * _RPT, _RPT)],
                    out_hbm.at[c, pl.ds(s * _RPT, _RPT)])


def _hop_pipeline(src, idxr, idxc, rows_a, rows_b, dst, sga, sgb):
    pltpu.async_copy(src.at[idxr.at[0]], rows_a, sga)
    pltpu.async_copy(src.at[idxr.at[1]], rows_b, sgb)

    def body(g, carry):
        pltpu.make_async_copy(src.at[idxr.at[0]], rows_a, sga).wait()
        pltpu.sync_copy(rows_a, dst.at[idxc.at[2 * g]], add=True)
        pltpu.async_copy(src.at[idxr.at[2 * g + 2]], rows_a, sga)
        pltpu.make_async_copy(src.at[idxr.at[1]], rows_b, sgb).wait()
        pltpu.sync_copy(rows_b, dst.at[idxc.at[2 * g + 1]], add=True)
        pltpu.async_copy(src.at[idxr.at[2 * g + 3]], rows_b, sgb)
        return carry

    lax.fori_loop(0, _NCH // 2 - 1, body, 0)
    pltpu.make_async_copy(src.at[idxr.at[0]], rows_a, sga).wait()
    pltpu.sync_copy(rows_a, dst.at[idxc.at[_NCH - 2]], add=True)
    pltpu.make_async_copy(src.at[idxr.at[1]], rows_b, sgb).wait()
    pltpu.sync_copy(rows_b, dst.at[idxc.at[_NCH - 1]], add=True)


@functools.partial(
    pl.kernel,
    mesh=_sc_mesh,
    compiler_params=_sc_params,
    out_type=jax.ShapeDtypeStruct((_NC, _NP, _DH), jnp.bfloat16),
    scratch_types=[
        pltpu.VMEM((_NCH, _CHUNK), jnp.int32),
        pltpu.VMEM((_NCH, _CHUNK), jnp.int32),
        pltpu.VMEM((_CHUNK, _DH), jnp.bfloat16),
        pltpu.VMEM((_CHUNK, _DH), jnp.bfloat16),
        pltpu.VMEM((_RPT, _DH), jnp.bfloat16),
        pltpu.VMEM((_RPT,), jnp.float32),
        pltpu.VMEM_SHARED((_NP, _DH), jnp.bfloat16),
        pltpu.VMEM_SHARED((_NP, _DH), jnp.bfloat16),
        pltpu.SemaphoreType.DMA,
        pltpu.SemaphoreType.DMA,
    ],
)
def _layer(u_hbm, dinv_hbm, row_hbm, col_hbm, out_hbm, idxr, idxc,
           rows_a, rows_b, stage, dv, acc1, acc2, sga, sgb):
    c = lax.axis_index("c")
    s = lax.axis_index("s")
    rb = s * _RPT
    pltpu.sync_copy(u_hbm.at[c, pl.ds(rb, _RPT)], acc1.at[pl.ds(rb, _RPT)])
    pltpu.sync_copy(row_hbm.at[pl.ds(s * _NCH, _NCH)], idxr)
    pltpu.sync_copy(col_hbm.at[pl.ds(s * _NCH, _NCH)], idxc)
    plsc.subcore_barrier()

    _hop_pipeline(u_hbm.at[c], idxr, idxc, rows_a, rows_b, acc1, sga, sgb)
    plsc.subcore_barrier()

    pltpu.sync_copy(dinv_hbm.at[pl.ds(rb, _RPT)], dv)
    pltpu.sync_copy(acc1.at[pl.ds(rb, _RPT)], stage)

    def scale(m, carry):
        dvec = dv[pl.ds(m * 16, 16)]
        for j in range(16):
            n = m * 16 + j
            d2 = dvec[j] * dvec[j]
            va, vb = plsc.unpack(stage[n, :],
                                 format=plsc.PackFormat.INTERLEAVED)
            stage[n, :] = plsc.pack(va * d2, vb * d2,
                                    format=plsc.PackFormat.INTERLEAVED)
        return carry

    lax.fori_loop(0, _RPT // 16, scale, 0)
    pltpu.sync_copy(stage, acc1.at[pl.ds(rb, _RPT)])
    pltpu.sync_copy(stage, acc2.at[pl.ds(rb, _RPT)])
    plsc.subcore_barrier()

    _hop_pipeline(acc1, idxr, idxc, rows_a, rows_b, acc2, sga, sgb)
    plsc.subcore_barrier()
    pltpu.sync_copy(acc2.at[pl.ds(rb, _RPT)], out_hbm.at[c, pl.ds(rb, _RPT)])


_ROWS_B = 1024
_GRID = _NP // _ROWS_B


def _entry_body(d0_ref, d1_ref, x_ref, w1_ref, dinv_ref, dinvf_ref, u_ref):
    deg = d0_ref[0, :, :1] + d1_ref[0, :, :1] - 1.0
    dinv = lax.rsqrt(deg)
    dinv_ref[:] = dinv
    dinvf_ref[:] = dinv.reshape(_ROWS_B // 128, 128)
    for c in range(_NC):
        u_ref[c] = (jnp.dot(x_ref[:], w1_ref[c],
                            preferred_element_type=jnp.float32)
                    * dinv).astype(jnp.bfloat16)


def _entry(degp, xp, W1):
    return pl.pallas_call(
        _entry_body,
        grid=(_GRID,),
        in_specs=[
            pl.BlockSpec((1, _ROWS_B, 16), lambda i: (0, i, 0)),
            pl.BlockSpec((1, _ROWS_B, 16), lambda i: (1, i, 0)),
            pl.BlockSpec((_ROWS_B, 128), lambda i: (i, 0)),
            pl.BlockSpec((_NC, 128, _DH), lambda i: (0, 0, 0)),
        ],
        out_specs=[
            pl.BlockSpec((_ROWS_B, 1), lambda i: (i, 0)),
            pl.BlockSpec((_ROWS_B // 128, 128), lambda i: (i, 0)),
            pl.BlockSpec((_NC, _ROWS_B, _DH), lambda i: (0, i, 0)),
        ],
        out_shape=[
            jax.ShapeDtypeStruct((_NP, 1), jnp.float32),
            jax.ShapeDtypeStruct((_NP // 128, 128), jnp.float32),
            jax.ShapeDtypeStruct((_NC, _NP, _DH), jnp.bfloat16),
        ],
    )(degp, degp, xp, W1)


def _mid_body(z_ref, dinv_ref, b1_ref, w2_ref, u_ref):
    dinv = dinv_ref[:]
    z = jnp.concatenate([z_ref[0], z_ref[1]],
                        axis=1).astype(jnp.float32)
    h = jnp.maximum(z * dinv + b1_ref[:], 0.0)
    for c in range(_NC):
        u_ref[c] = (jnp.dot(h, w2_ref[c],
                            preferred_element_type=jnp.float32)
                    * dinv).astype(jnp.bfloat16)


def _mid(z3, dinv, b1, W2p):
    return pl.pallas_call(
        _mid_body,
        grid=(_GRID,),
        in_specs=[
            pl.BlockSpec((_NC, _ROWS_B, _DH), lambda i: (0, i, 0)),
            pl.BlockSpec((_ROWS_B, 1), lambda i: (i, 0)),
            pl.BlockSpec((1, 2 * _DH), lambda i: (0, 0)),
            pl.BlockSpec((_NC, 2 * _DH, _DH), lambda i: (0, 0, 0)),
        ],
        out_specs=pl.BlockSpec((_NC, _ROWS_B, _DH), lambda i: (0, i, 0)),
        out_shape=jax.ShapeDtypeStruct((_NC, _NP, _DH), jnp.bfloat16),
    )(z3, dinv, b1, W2p)


def _final_body(z_ref, dinv_ref, b2_ref, out_ref):
    z = jnp.concatenate([z_ref[0], z_ref[1]],
                        axis=1).astype(jnp.float32)
    t = z * dinv_ref[:] + b2_ref[:]
    t = t[:, :40]
    m = jnp.max(t, axis=1, keepdims=True)
    e = t - m
    out_ref[:] = e - jnp.log(jnp.sum(jnp.exp(e), axis=1, keepdims=True))


def _final(z3, dinv, b2p):
    return pl.pallas_call(
        _final_body,
        grid=(_GRID,),
        in_specs=[
            pl.BlockSpec((_NC, _ROWS_B, _DH), lambda i: (0, i, 0)),
            pl.BlockSpec((_ROWS_B, 1), lambda i: (i, 0)),
            pl.BlockSpec((1, 2 * _DH), lambda i: (0, 0)),
        ],
        out_specs=pl.BlockSpec((_ROWS_B, 40), lambda i: (i, 0)),
        out_shape=jax.ShapeDtypeStruct((_NP, 40), jnp.float32),
    )(z3, dinv, b2p)


def kernel(x, edge_index, W1, b1, W2, b2):
    epad = _N + jnp.arange(_EPAD - _E, dtype=jnp.int32) % (_NP - _N)
    row = jnp.concatenate([edge_index[0], epad]).reshape(-1, _CHUNK)
    col = jnp.concatenate([edge_index[1], epad]).reshape(-1, _CHUNK)
    xp = jnp.pad(x, ((0, _NP - _N), (0, 0)))
    ones16 = jnp.ones((_NP, 16), jnp.float32)
    W2f = jnp.pad(W2, ((0, 0), (0, 2 * _DH - 40)))
    W1c = jnp.stack([W1[:, :_DH], W1[:, _DH:]])
    W2p = jnp.stack([W2f[:, :_DH], W2f[:, _DH:]])
    b1r = b1.reshape(1, 2 * _DH)
    b2r = jnp.pad(b2, (0, 2 * _DH - 40)).reshape(1, 2 * _DH)

    degp = _degrees(ones16, col)
    dinv, dinvf, u1 = _entry(degp, xp, W1c)
    dinv1 = dinvf.reshape(_NP)
    z1 = _layer(u1, dinv1, row, col)
    u2 = _mid(z1, dinv, b1r, W2p)
    z2 = _layer(u2, dinv1, row, col)
    out = _final(z2, dinv, b2r)
    return out[:_N]
